# baseline (speedup 1.0000x reference)
import functools

import jax
import jax.numpy as jnp
from jax.experimental import pallas as pl
from jax.experimental.pallas import tpu as pltpu


def _round_up(v, m):
    return (v + m - 1) // m * m


def _gnn_kernel(num_convs,
                idx_ref,
                a_ref, x_ref, w_ref, b_ref,
                wl_ref, bl_ref,
                o_ref):
    a = a_ref[...]
    n = a.shape[0]
    m = o_ref.shape[0]

    ones = jnp.ones((n, 128), jnp.bfloat16)
    deg = jnp.dot(a, ones, preferred_element_type=jnp.float32)[:, :1] + 1.0
    dinv = jax.lax.rsqrt(deg)

    h = x_ref[...]
    for l in range(num_convs):
        xw = jnp.dot(h.astype(jnp.bfloat16), w_ref[l],
                     preferred_element_type=jnp.float32)
        y = xw * dinv
        agg = jnp.dot(a, y.astype(jnp.bfloat16),
                      preferred_element_type=jnp.float32) + y
        h = jnp.maximum(agg * dinv + b_ref[l], 0.0)

    hb = h.astype(jnp.bfloat16)
    hs = jnp.dot(hb, wl_ref[0], preferred_element_type=jnp.float32)
    hd = jnp.dot(hb, wl_ref[1], preferred_element_type=jnp.float32)

    rows = jax.lax.broadcasted_iota(jnp.int32, (n, m), 0)
    oh_s = (rows == idx_ref[0:1, :]).astype(jnp.float32)
    oh_d = (rows == idx_ref[1:2, :]).astype(jnp.float32)
    contract = (((0,), (0,)), ((), ()))
    gs = jax.lax.dot_general(oh_s, hs, contract,
                             preferred_element_type=jnp.float32)
    gd = jax.lax.dot_general(oh_d, hd, contract,
                             preferred_element_type=jnp.float32)

    z = gs + gd + bl_ref[...]
    o_ref[...] = 0.5 * (jnp.tanh(0.5 * z) + 1.0)


def _forward(params, x, g_edge_index, index01):
    n, d = x.shape
    convs = params["convs"]
    num_convs = len(convs)
    hdim = convs[-1][0].shape[1]
    out_dim = params["linear_w"].shape[0]
    m = index01.shape[0]

    LANE, SUB_BF16 = 128, 16
    n_pad = _round_up(n, SUB_BF16)
    p = _round_up(max(d, hdim), LANE)
    o_pad = _round_up(out_dim, LANE)
    m_pad = _round_up(m, LANE)

    src, dst = g_edge_index[0], g_edge_index[1]
    a_p = (jnp.zeros((n_pad, n_pad), jnp.bfloat16)
           .at[dst, src].add(jnp.bfloat16(1.0)))

    x_p = (jnp.zeros((n_pad, p), jnp.float32).at[:n, :d].set(x)
           .astype(jnp.bfloat16))

    w_stack = jnp.zeros((num_convs, p, p), jnp.float32)
    b_stack = jnp.zeros((num_convs, 1, p), jnp.float32)
    for l, (w, b) in enumerate(convs):
        w_stack = w_stack.at[l, :w.shape[0], :w.shape[1]].set(w)
        b_stack = b_stack.at[l, 0, :b.shape[0]].set(b)
    w_stack = w_stack.astype(jnp.bfloat16)

    wl_t = params["linear_w"].T
    wl_stack = jnp.zeros((2, p, o_pad), jnp.float32)
    wl_stack = wl_stack.at[0, :hdim, :out_dim].set(wl_t[:hdim])
    wl_stack = wl_stack.at[1, :hdim, :out_dim].set(wl_t[hdim:])
    wl_stack = wl_stack.astype(jnp.bfloat16)
    bl_p = jnp.zeros((1, o_pad), jnp.float32).at[0, :out_dim].set(
        params["linear_b"])

    idx = jnp.zeros((8, m_pad), jnp.int32)
    idx = idx.at[0, :m].set(src[index01].astype(jnp.int32))
    idx = idx.at[1, :m].set(dst[index01].astype(jnp.int32))

    out_p = pl.pallas_call(
        functools.partial(_gnn_kernel, num_convs),
        out_shape=jax.ShapeDtypeStruct((m_pad, o_pad), jnp.float32),
    )(idx, a_p, x_p, w_stack, b_stack, wl_stack, bl_p)

    return out_p[:m, :out_dim][None]


def kernel(x, w1, b1, w2, b2, linear_w, linear_b,
           g_edge_index, lg_edge_index, index01):
    del lg_edge_index
    params = {
        "convs": [(w1, b1), (w2, b2)],
        "linear_w": linear_w,
        "linear_b": linear_b,
    }
    return _forward(params, x, g_edge_index, index01)


# speedup vs baseline: 3.0874x; 3.0874x over previous
import functools

import jax
import jax.numpy as jnp
from jax.experimental import pallas as pl
from jax.experimental.pallas import tpu as pltpu


def _round_up(v, m):
    return (v + m - 1) // m * m


def _gnn_kernel(num_convs,
                idx_ref,
                a_ref, x_ref, w_ref, b_ref,
                wl_ref, bl_ref,
                o_ref):
    a = a_ref[...]
    n = a.shape[0]
    m = o_ref.shape[0]

    ones = jnp.ones((n, 128), jnp.bfloat16)
    deg = jnp.dot(a, ones, preferred_element_type=jnp.float32)[:, :1] + 1.0
    dinv = jax.lax.rsqrt(deg)

    h = x_ref[...]
    for l in range(num_convs):
        xw = jnp.dot(h.astype(jnp.bfloat16), w_ref[l],
                     preferred_element_type=jnp.float32)
        y = xw * dinv
        agg = jnp.dot(a, y.astype(jnp.bfloat16),
                      preferred_element_type=jnp.float32) + y
        h = jnp.maximum(agg * dinv + b_ref[l], 0.0)

    hb = h.astype(jnp.bfloat16)
    hs = jnp.dot(hb, wl_ref[0], preferred_element_type=jnp.float32)
    hd = jnp.dot(hb, wl_ref[1], preferred_element_type=jnp.float32)

    rows = jax.lax.broadcasted_iota(jnp.int32, (n, m), 0)
    oh_s = (rows == idx_ref[0:1, :]).astype(jnp.float32)
    oh_d = (rows == idx_ref[1:2, :]).astype(jnp.float32)
    contract = (((0,), (0,)), ((), ()))
    gs = jax.lax.dot_general(oh_s, hs, contract,
                             preferred_element_type=jnp.float32)
    gd = jax.lax.dot_general(oh_d, hd, contract,
                             preferred_element_type=jnp.float32)

    z = gs + gd + bl_ref[...]
    o_ref[...] = 0.5 * (jnp.tanh(0.5 * z) + 1.0)


def _forward(params, x, g_edge_index, index01):
    n, d = x.shape
    convs = params["convs"]
    num_convs = len(convs)
    hdim = convs[-1][0].shape[1]
    out_dim = params["linear_w"].shape[0]
    m = index01.shape[0]

    LANE, SUB_BF16 = 128, 16
    n_pad = _round_up(n, SUB_BF16)
    p = _round_up(max(d, hdim), LANE)
    o_pad = _round_up(out_dim, LANE)
    m_pad = _round_up(m, LANE)

    src, dst = g_edge_index[0], g_edge_index[1]
    lin = dst * n_pad + src
    a_raw = (jnp.zeros((n_pad * n_pad,), jnp.float32)
             .at[lin].add(1.0, mode="promise_in_bounds")
             .reshape(n_pad, n_pad))
    a_p = a_raw.astype(jnp.bfloat16)

    x_p = (jnp.zeros((n_pad, p), jnp.float32).at[:n, :d].set(x)
           .astype(jnp.bfloat16))

    w_stack = jnp.zeros((num_convs, p, p), jnp.float32)
    b_stack = jnp.zeros((num_convs, 1, p), jnp.float32)
    for l, (w, b) in enumerate(convs):
        w_stack = w_stack.at[l, :w.shape[0], :w.shape[1]].set(w)
        b_stack = b_stack.at[l, 0, :b.shape[0]].set(b)
    w_stack = w_stack.astype(jnp.bfloat16)

    wl_t = params["linear_w"].T
    wl_stack = jnp.zeros((2, p, o_pad), jnp.float32)
    wl_stack = wl_stack.at[0, :hdim, :out_dim].set(wl_t[:hdim])
    wl_stack = wl_stack.at[1, :hdim, :out_dim].set(wl_t[hdim:])
    wl_stack = wl_stack.astype(jnp.bfloat16)
    bl_p = jnp.zeros((1, o_pad), jnp.float32).at[0, :out_dim].set(
        params["linear_b"])

    idx = jnp.zeros((8, m_pad), jnp.int32)
    idx = idx.at[0, :m].set(src[index01].astype(jnp.int32))
    idx = idx.at[1, :m].set(dst[index01].astype(jnp.int32))

    out_p = pl.pallas_call(
        functools.partial(_gnn_kernel, num_convs),
        out_shape=jax.ShapeDtypeStruct((m_pad, o_pad), jnp.float32),
    )(idx, a_p, x_p, w_stack, b_stack, wl_stack, bl_p)

    return out_p[:m, :out_dim][None]


def kernel(x, w1, b1, w2, b2, linear_w, linear_b,
           g_edge_index, lg_edge_index, index01):
    del lg_edge_index
    params = {
        "convs": [(w1, b1), (w2, b2)],
        "linear_w": linear_w,
        "linear_b": linear_b,
    }
    return _forward(params, x, g_edge_index, index01)
